# Initial kernel scaffold; baseline (speedup 1.0000x reference)
import jax
import jax.numpy as jnp
from jax.experimental import pallas as pl


def kernel(x, table):
    raise NotImplementedError("write your pallas kernel here")


# speedup vs baseline: 1.5182x; 1.5182x over previous
import functools
import math

import jax
import jax.numpy as jnp
from jax import lax
from jax.experimental import pallas as pl
from jax.experimental.pallas import tpu as pltpu
from jax.experimental.pallas import tpu_sc as plsc

VOCAB = 100000
D = 1024
SCALE = math.sqrt(D)

NC, NS, L = 2, 16, 16
NW = NC * NS
B = 4 * 4096
B_PER_W = B // NW
CHUNK = 16
NBUF = 4
NCHUNK = B_PER_W // CHUNK

_mesh = plsc.VectorSubcoreMesh(core_axis_name="c", subcore_axis_name="s")


@functools.partial(
    pl.kernel,
    out_type=jax.ShapeDtypeStruct((B, D), jnp.float32),
    mesh=_mesh,
    scratch_types=(
        pltpu.VMEM((B_PER_W,), jnp.int32),
        [pltpu.VMEM((CHUNK, D), jnp.float32) for _ in range(NBUF)],
        [pltpu.SemaphoreType.DMA for _ in range(NBUF)],
        [pltpu.SemaphoreType.DMA for _ in range(NBUF)],
    ),
)
def _emb_kernel(x_hbm, table_hbm, out_hbm, idx_v, bufs, gsems, ssems):
    wid = lax.axis_index("s") * NC + lax.axis_index("c")
    base = wid * B_PER_W

    pltpu.sync_copy(x_hbm.at[pl.ds(base, B_PER_W)], idx_v)

    def start_gather(c):
        b = c % NBUF
        return pltpu.async_copy(
            table_hbm.at[idx_v.at[pl.ds(c * CHUNK, CHUNK)]], bufs[b], gsems[b])

    gathers = {}
    stores = {}
    for c in range(min(2, NCHUNK)):
        gathers[c] = start_gather(c)

    for c in range(NCHUNK):
        b = c % NBUF
        gathers[c].wait()

        buf = bufs[b]

        @pl.loop(0, CHUNK)
        def _row(r):
            @pl.loop(0, D // L, unroll=8)
            def _col(j):
                sl = pl.ds(j * L, L)
                buf[r, sl] = buf[r, sl] * SCALE

        stores[c] = pltpu.async_copy(
            buf, out_hbm.at[pl.ds(base + c * CHUNK, CHUNK)], ssems[b])

        nc = c + 2
        if nc < NCHUNK:
            if nc - NBUF >= 0:
                stores[nc - NBUF].wait()
            gathers[nc] = start_gather(nc)

    for c in range(max(0, NCHUNK - NBUF), NCHUNK):
        stores[c].wait()


def kernel(x, table):
    flat = x.reshape(-1).astype(jnp.int32)
    out = _emb_kernel(flat, table)
    return out.reshape(x.shape + (D,))


# speedup vs baseline: 1.5591x; 1.0269x over previous
import functools
import math

import jax
import jax.numpy as jnp
from jax import lax
from jax.experimental import pallas as pl
from jax.experimental.pallas import tpu as pltpu
from jax.experimental.pallas import tpu_sc as plsc

VOCAB = 100000
D = 1024
SCALE = math.sqrt(D)

NC, NS, L = 2, 16, 16
NW = NC * NS
B = 4 * 4096
B_PER_W = B // NW
CHUNK = 16
NBUF = 6
NCHUNK = B_PER_W // CHUNK

_mesh = plsc.VectorSubcoreMesh(core_axis_name="c", subcore_axis_name="s")


@functools.partial(
    pl.kernel,
    out_type=jax.ShapeDtypeStruct((B, D), jnp.float32),
    mesh=_mesh,
    scratch_types=(
        pltpu.VMEM((B_PER_W,), jnp.int32),
        [pltpu.VMEM((CHUNK, D), jnp.float32) for _ in range(NBUF)],
        [pltpu.SemaphoreType.DMA for _ in range(NBUF)],
        [pltpu.SemaphoreType.DMA for _ in range(NBUF)],
    ),
)
def _emb_kernel(x_hbm, table_hbm, out_hbm, idx_v, bufs, gsems, ssems):
    wid = lax.axis_index("s") * NC + lax.axis_index("c")
    base = wid * B_PER_W

    pltpu.sync_copy(x_hbm.at[pl.ds(base, B_PER_W)], idx_v)

    def start_gather(c):
        b = c % NBUF
        return pltpu.async_copy(
            table_hbm.at[idx_v.at[pl.ds(c * CHUNK, CHUNK)]], bufs[b], gsems[b])

    gathers = {}
    stores = {}
    for c in range(min(4, NCHUNK)):
        gathers[c] = start_gather(c)

    for c in range(NCHUNK):
        b = c % NBUF
        gathers[c].wait()

        buf = bufs[b]

        @pl.loop(0, CHUNK)
        def _row(r):
            @pl.loop(0, D // L, unroll=8)
            def _col(j):
                sl = pl.ds(j * L, L)
                buf[r, sl] = buf[r, sl] * SCALE

        stores[c] = pltpu.async_copy(
            buf, out_hbm.at[pl.ds(base + c * CHUNK, CHUNK)], ssems[b])

        nc = c + 4
        if nc < NCHUNK:
            if nc - NBUF >= 0:
                stores[nc - NBUF].wait()
            gathers[nc] = start_gather(nc)

    for c in range(max(0, NCHUNK - NBUF), NCHUNK):
        stores[c].wait()


def kernel(x, table):
    flat = x.reshape(-1).astype(jnp.int32)
    out = _emb_kernel(flat, table)
    return out.reshape(x.shape + (D,))


# speedup vs baseline: 1.6510x; 1.0589x over previous
import functools
import math

import jax
import jax.numpy as jnp
from jax import lax
from jax.experimental import pallas as pl
from jax.experimental.pallas import tpu as pltpu
from jax.experimental.pallas import tpu_sc as plsc

VOCAB = 100000
D = 1024
SCALE = math.sqrt(D)

NC, NS, L = 2, 16, 16
NW = NC * NS
XROWS, XCOLS = 4, 4096
B = XROWS * XCOLS
B_PER_W = B // NW
W_PER_XROW = XCOLS // B_PER_W
CHUNK = 16
NBUF = 6
DIST = 4
NCHUNK = B_PER_W // CHUNK

_mesh = plsc.VectorSubcoreMesh(core_axis_name="c", subcore_axis_name="s")


@functools.partial(
    pl.kernel,
    out_type=jax.ShapeDtypeStruct((B, D), jnp.float32),
    mesh=_mesh,
    scratch_types=(
        pltpu.VMEM((B_PER_W,), jnp.int32),
        [pltpu.VMEM((CHUNK, D), jnp.float32) for _ in range(NBUF)],
        [pltpu.SemaphoreType.DMA for _ in range(NBUF)],
        [pltpu.SemaphoreType.DMA for _ in range(NBUF)],
    ),
)
def _emb_kernel(x_hbm, table_hbm, out_hbm, idx_v, bufs, gsems, ssems):
    wid = lax.axis_index("s") * NC + lax.axis_index("c")
    base = wid * B_PER_W

    xr = wid // W_PER_XROW
    xc = (wid % W_PER_XROW) * B_PER_W
    pltpu.sync_copy(x_hbm.at[xr, pl.ds(xc, B_PER_W)], idx_v)

    def start_gather(c):
        b = c % NBUF
        return pltpu.async_copy(
            table_hbm.at[idx_v.at[pl.ds(c * CHUNK, CHUNK)]], bufs[b], gsems[b])

    gathers = {}
    stores = {}
    for c in range(min(DIST, NCHUNK)):
        gathers[c] = start_gather(c)

    for c in range(NCHUNK):
        b = c % NBUF
        gathers[c].wait()

        buf = bufs[b]

        @functools.partial(plsc.parallel_loop, 0, CHUNK * (D // L), unroll=8)
        def _scale(k):
            r = k // (D // L)
            sl = pl.ds((k % (D // L)) * L, L)
            buf[r, sl] = buf[r, sl] * SCALE

        stores[c] = pltpu.async_copy(
            buf, out_hbm.at[pl.ds(base + c * CHUNK, CHUNK)], ssems[b])

        nc = c + DIST
        if nc < NCHUNK:
            if nc - NBUF >= 0:
                stores[nc - NBUF].wait()
            gathers[nc] = start_gather(nc)

    for c in range(max(0, NCHUNK - NBUF), NCHUNK):
        stores[c].wait()


def kernel(x, table):
    out = _emb_kernel(x, table)
    return out.reshape(XROWS, XCOLS, D)
